# baseline (speedup 1.0000x reference)
import functools

import jax
import jax.numpy as jnp
from jax.experimental import pallas as pl
from jax.experimental.pallas import tpu as pltpu


def _proj_kernel(nr_grid, h_ref, w_ref, a_ref,
                 whext_ref, s_ref, t_ref, ms_ref, mt_ref, sumwh_ref,
                 ms_s, mt_s, sum_s):
    r = pl.program_id(0)
    f = w_ref.shape[1]
    wh = jnp.dot(h_ref[...], w_ref[...], preferred_element_type=jnp.float32)
    s = jnp.dot(wh, a_ref[:f, :], preferred_element_type=jnp.float32)
    t = jnp.dot(wh, a_ref[f:, :], preferred_element_type=jnp.float32)
    whext_ref[...] = jnp.concatenate(
        [wh, jnp.ones((wh.shape[0], 1), jnp.float32)], axis=1)
    s_ref[...] = s
    t_ref[...] = t

    bs = jnp.max(s, axis=(0, 1), keepdims=True)
    bt = jnp.max(t, axis=(0, 1), keepdims=True)
    bsum = jnp.sum(wh, axis=0, keepdims=True)

    @pl.when(r == 0)
    def _():
        ms_s[...] = bs
        mt_s[...] = bt
        sum_s[...] = bsum

    @pl.when(r > 0)
    def _():
        ms_s[...] = jnp.maximum(ms_s[...], bs)
        mt_s[...] = jnp.maximum(mt_s[...], bt)
        sum_s[...] = sum_s[...] + bsum

    @pl.when(r == nr_grid - 1)
    def _():
        ms_ref[...] = ms_s[...]
        mt_ref[...] = mt_s[...]
        sumwh_ref[...] = sum_s[...]


def _att_accumulate(n, bc, c, adj_ref, s_ref, t_ref, tc_ref, ms_ref, mt_ref,
                    whext_ref, acc):
    msv = ms_ref[...]
    mtv = mt_ref[...]
    cm = msv + mtv
    cmax = jnp.where(cm > 0, cm, 0.2 * cm)
    s = s_ref[...]
    t = t_ref[...]
    tc = tc_ref[...]
    u1 = jnp.exp(s + mtv - cmax)
    u2 = jnp.exp(0.2 * (s + mtv) - cmax)
    wj = jnp.exp(0.8 * (t - mtv))
    colid = c * bc + jax.lax.broadcasted_iota(jnp.int32, tc.shape, 0)
    v2c = jnp.where(colid < n, jnp.exp(0.2 * (tc - mtv)), 0.0)
    p = jnp.maximum(u1 * wj, u2)
    p = jnp.where(adj_ref[...] > 0, p, 0.0)
    acc[...] += jnp.dot(p, whext_ref[...] * v2c,
                        preferred_element_type=jnp.float32)


def _att1_kernel(n, nr_grid, nc, br, bc,
                 adj_ref, s_ref, t_ref, tc_ref, ms_ref, mt_ref, whext_ref,
                 sumwh_ref, w2_ref, a2_ref,
                 whext2_ref, s2_ref, t2_ref, ms2_ref, mt2_ref, sumwh2_ref,
                 acc, ms_s, mt_s, sum_s):
    c = pl.program_id(1)

    @pl.when(c == 0)
    def _():
        acc[...] = jnp.zeros_like(acc)

    _att_accumulate(n, bc, c, adj_ref, s_ref, t_ref, tc_ref, ms_ref, mt_ref,
                    whext_ref, acc)

    @pl.when(c == nc - 1)
    def _():
        r = pl.program_id(0)
        f = sumwh_ref.shape[1]
        accv = acc[...]
        num = accv[:, :f]
        den = accv[:, f:f + 1]
        fb = sumwh_ref[...] * (1.0 / n)
        h1 = jnp.where(den > 0, num / den, fb)
        h1 = jnp.maximum(h1, 0.0)
        rowid = r * br + jax.lax.broadcasted_iota(jnp.int32, h1.shape, 0)
        h1 = jnp.where(rowid < n, h1, 0.0)

        f2 = w2_ref.shape[1]
        wh2 = jnp.dot(h1, w2_ref[...], preferred_element_type=jnp.float32)
        s2 = jnp.dot(wh2, a2_ref[:f2, :], preferred_element_type=jnp.float32)
        t2 = jnp.dot(wh2, a2_ref[f2:, :], preferred_element_type=jnp.float32)
        whext2_ref[...] = jnp.concatenate(
            [wh2, jnp.ones((wh2.shape[0], 1), jnp.float32)], axis=1)
        s2_ref[...] = s2
        t2_ref[...] = t2

        bs = jnp.max(s2, axis=(0, 1), keepdims=True)
        bt = jnp.max(t2, axis=(0, 1), keepdims=True)
        bsum = jnp.sum(wh2, axis=0, keepdims=True)

        @pl.when(r == 0)
        def _():
            ms_s[...] = bs
            mt_s[...] = bt
            sum_s[...] = bsum

        @pl.when(r > 0)
        def _():
            ms_s[...] = jnp.maximum(ms_s[...], bs)
            mt_s[...] = jnp.maximum(mt_s[...], bt)
            sum_s[...] = sum_s[...] + bsum

        @pl.when(r == nr_grid - 1)
        def _():
            ms2_ref[...] = ms_s[...]
            mt2_ref[...] = mt_s[...]
            sumwh2_ref[...] = sum_s[...]


def _att2_kernel(n, nc, bc,
                 adj_ref, s_ref, t_ref, tc_ref, ms_ref, mt_ref, whext_ref,
                 sumwh_ref, wd_ref, bd_ref, z_ref, acc):
    c = pl.program_id(1)

    @pl.when(c == 0)
    def _():
        acc[...] = jnp.zeros_like(acc)

    _att_accumulate(n, bc, c, adj_ref, s_ref, t_ref, tc_ref, ms_ref, mt_ref,
                    whext_ref, acc)

    @pl.when(c == nc - 1)
    def _():
        f = sumwh_ref.shape[1]
        accv = acc[...]
        num = accv[:, :f]
        den = accv[:, f:f + 1]
        fb = sumwh_ref[...] * (1.0 / n)
        mu = jnp.where(den > 0, num / den, fb)
        z_ref[...] = jnp.dot(mu, wd_ref[...],
                             preferred_element_type=jnp.float32) + bd_ref[...]


def kernel(x, adj, W1, a1, W2, a2, W3, a3, w2_w, w2_b):
    del W3, a3
    n, din = x.shape
    h1f = W1.shape[1]
    h2f = W2.shape[1]
    br, bc = 256, 1024
    nc = pl.cdiv(n, bc)
    npad = nc * bc
    nr = npad // br

    x_p = jnp.pad(x, ((0, npad - n), (0, 0)))

    bp = 512
    np_grid = npad // bp
    whext1, s1, t1, ms1, mt1, sumwh1 = pl.pallas_call(
        functools.partial(_proj_kernel, np_grid),
        grid=(np_grid,),
        in_specs=[
            pl.BlockSpec((bp, din), lambda r: (r, 0)),
            pl.BlockSpec((din, h1f), lambda r: (0, 0)),
            pl.BlockSpec((2 * h1f, 1), lambda r: (0, 0)),
        ],
        out_specs=[
            pl.BlockSpec((bp, h1f + 1), lambda r: (r, 0)),
            pl.BlockSpec((bp, 1), lambda r: (r, 0)),
            pl.BlockSpec((bp, 1), lambda r: (r, 0)),
            pl.BlockSpec((1, 1), lambda r: (0, 0)),
            pl.BlockSpec((1, 1), lambda r: (0, 0)),
            pl.BlockSpec((1, h1f), lambda r: (0, 0)),
        ],
        out_shape=[
            jax.ShapeDtypeStruct((npad, h1f + 1), jnp.float32),
            jax.ShapeDtypeStruct((npad, 1), jnp.float32),
            jax.ShapeDtypeStruct((npad, 1), jnp.float32),
            jax.ShapeDtypeStruct((1, 1), jnp.float32),
            jax.ShapeDtypeStruct((1, 1), jnp.float32),
            jax.ShapeDtypeStruct((1, h1f), jnp.float32),
        ],
        scratch_shapes=[
            pltpu.VMEM((1, 1), jnp.float32),
            pltpu.VMEM((1, 1), jnp.float32),
            pltpu.VMEM((1, h1f), jnp.float32),
        ],
    )(x_p, W1, a1)

    whext2, s2, t2, ms2, mt2, sumwh2 = pl.pallas_call(
        functools.partial(_att1_kernel, n, nr, nc, br, bc),
        grid=(nr, nc),
        in_specs=[
            pl.BlockSpec((br, bc), lambda r, c: (r, c)),
            pl.BlockSpec((br, 1), lambda r, c: (r, 0)),
            pl.BlockSpec((1, bc), lambda r, c: (0, c)),
            pl.BlockSpec((bc, 1), lambda r, c: (c, 0)),
            pl.BlockSpec((1, 1), lambda r, c: (0, 0)),
            pl.BlockSpec((1, 1), lambda r, c: (0, 0)),
            pl.BlockSpec((bc, h1f + 1), lambda r, c: (c, 0)),
            pl.BlockSpec((1, h1f), lambda r, c: (0, 0)),
            pl.BlockSpec((h1f, h2f), lambda r, c: (0, 0)),
            pl.BlockSpec((2 * h2f, 1), lambda r, c: (0, 0)),
        ],
        out_specs=[
            pl.BlockSpec((br, h2f + 1), lambda r, c: (r, 0)),
            pl.BlockSpec((br, 1), lambda r, c: (r, 0)),
            pl.BlockSpec((br, 1), lambda r, c: (r, 0)),
            pl.BlockSpec((1, 1), lambda r, c: (0, 0)),
            pl.BlockSpec((1, 1), lambda r, c: (0, 0)),
            pl.BlockSpec((1, h2f), lambda r, c: (0, 0)),
        ],
        out_shape=[
            jax.ShapeDtypeStruct((npad, h2f + 1), jnp.float32),
            jax.ShapeDtypeStruct((npad, 1), jnp.float32),
            jax.ShapeDtypeStruct((npad, 1), jnp.float32),
            jax.ShapeDtypeStruct((1, 1), jnp.float32),
            jax.ShapeDtypeStruct((1, 1), jnp.float32),
            jax.ShapeDtypeStruct((1, h2f), jnp.float32),
        ],
        scratch_shapes=[
            pltpu.VMEM((br, h1f + 1), jnp.float32),
            pltpu.VMEM((1, 1), jnp.float32),
            pltpu.VMEM((1, 1), jnp.float32),
            pltpu.VMEM((1, h2f), jnp.float32),
        ],
    )(adj, s1, t1.reshape(1, npad), t1, ms1, mt1, whext1, sumwh1, W2, a2)

    z = pl.pallas_call(
        functools.partial(_att2_kernel, n, nc, bc),
        grid=(nr, nc),
        in_specs=[
            pl.BlockSpec((br, bc), lambda r, c: (r, c)),
            pl.BlockSpec((br, 1), lambda r, c: (r, 0)),
            pl.BlockSpec((1, bc), lambda r, c: (0, c)),
            pl.BlockSpec((bc, 1), lambda r, c: (c, 0)),
            pl.BlockSpec((1, 1), lambda r, c: (0, 0)),
            pl.BlockSpec((1, 1), lambda r, c: (0, 0)),
            pl.BlockSpec((bc, h2f + 1), lambda r, c: (c, 0)),
            pl.BlockSpec((1, h2f), lambda r, c: (0, 0)),
            pl.BlockSpec((h2f, din), lambda r, c: (0, 0)),
            pl.BlockSpec((1, din), lambda r, c: (0, 0)),
        ],
        out_specs=pl.BlockSpec((br, din), lambda r, c: (r, 0)),
        out_shape=jax.ShapeDtypeStruct((n, din), jnp.float32),
        scratch_shapes=[
            pltpu.VMEM((br, h2f + 1), jnp.float32),
        ],
    )(adj, s2, t2.reshape(1, npad), t2, ms2, mt2, whext2, sumwh2, w2_w,
      w2_b.reshape(1, din))

    return z


# speedup vs baseline: 1.0891x; 1.0891x over previous
import functools

import jax
import jax.numpy as jnp
from jax.experimental import pallas as pl
from jax.experimental.pallas import tpu as pltpu


def _proj_kernel(nr_grid, h_ref, w_ref, a_ref,
                 whext_ref, s_ref, t_ref, ms_ref, mt_ref, sumwh_ref,
                 ms_s, mt_s, sum_s):
    r = pl.program_id(0)
    f = w_ref.shape[1]
    wh = jnp.dot(h_ref[...], w_ref[...], preferred_element_type=jnp.float32)
    s = jnp.dot(wh, a_ref[:f, :], preferred_element_type=jnp.float32)
    t = jnp.dot(wh, a_ref[f:, :], preferred_element_type=jnp.float32)
    whext_ref[...] = jnp.concatenate(
        [wh, jnp.ones((wh.shape[0], 1), jnp.float32)], axis=1)
    s_ref[...] = s
    t_ref[...] = t

    bs = jnp.max(s, axis=(0, 1), keepdims=True)
    bt = jnp.max(t, axis=(0, 1), keepdims=True)
    bsum = jnp.sum(wh, axis=0, keepdims=True)

    @pl.when(r == 0)
    def _():
        ms_s[...] = bs
        mt_s[...] = bt
        sum_s[...] = bsum

    @pl.when(r > 0)
    def _():
        ms_s[...] = jnp.maximum(ms_s[...], bs)
        mt_s[...] = jnp.maximum(mt_s[...], bt)
        sum_s[...] = sum_s[...] + bsum

    @pl.when(r == nr_grid - 1)
    def _():
        ms_ref[...] = ms_s[...]
        mt_ref[...] = mt_s[...]
        sumwh_ref[...] = sum_s[...]


def _scale_kernel(n, brs, t_ref, ms_ref, mt_ref, whext_ref,
                  whs_ref, wj_ref):
    r = pl.program_id(0)
    mtv = mt_ref[...]
    del ms_ref
    t = t_ref[...]
    rowid = r * brs + jax.lax.broadcasted_iota(jnp.int32, t.shape, 0)
    v2 = jnp.where(rowid < n, jnp.exp(0.2 * (t - mtv)), 0.0)
    whs_ref[...] = whext_ref[...] * v2
    wj_ref[...] = jnp.exp(0.8 * (t - mtv))


def _att_accumulate(adj_ref, s_ref, wj_ref, ms_ref, mt_ref, whs_ref, acc):
    msv = ms_ref[...]
    mtv = mt_ref[...]
    cm = msv + mtv
    cmax = jnp.where(cm > 0, cm, 0.2 * cm)
    s = s_ref[...]
    wj = wj_ref[...]
    u1 = jnp.exp(s + mtv - cmax)
    u2 = jnp.exp(0.2 * (s + mtv) - cmax)
    p = jnp.maximum(u1 * wj, u2)
    p = jnp.where(adj_ref[...] > 0, p, 0.0)
    acc[...] += jnp.dot(p, whs_ref[...], preferred_element_type=jnp.float32)


def _att1_kernel(n, nr_grid, nc, br, bc,
                 adj_ref, s_ref, wj_ref, ms_ref, mt_ref, whs_ref,
                 sumwh_ref, w2_ref, a2_ref,
                 whext2_ref, s2_ref, t2_ref, ms2_ref, mt2_ref, sumwh2_ref,
                 acc, ms_s, mt_s, sum_s):
    c = pl.program_id(1)

    @pl.when(c == 0)
    def _():
        acc[...] = jnp.zeros_like(acc)

    _att_accumulate(adj_ref, s_ref, wj_ref, ms_ref, mt_ref, whs_ref, acc)

    @pl.when(c == nc - 1)
    def _():
        r = pl.program_id(0)
        f = sumwh_ref.shape[1]
        accv = acc[...]
        num = accv[:, :f]
        den = accv[:, f:f + 1]
        fb = sumwh_ref[...] * (1.0 / n)
        h1 = jnp.where(den > 0, num / den, fb)
        h1 = jnp.maximum(h1, 0.0)
        rowid = r * br + jax.lax.broadcasted_iota(jnp.int32, h1.shape, 0)
        h1 = jnp.where(rowid < n, h1, 0.0)

        f2 = w2_ref.shape[1]
        wh2 = jnp.dot(h1, w2_ref[...], preferred_element_type=jnp.float32)
        s2 = jnp.dot(wh2, a2_ref[:f2, :], preferred_element_type=jnp.float32)
        t2 = jnp.dot(wh2, a2_ref[f2:, :], preferred_element_type=jnp.float32)
        whext2_ref[...] = jnp.concatenate(
            [wh2, jnp.ones((wh2.shape[0], 1), jnp.float32)], axis=1)
        s2_ref[...] = s2
        t2_ref[...] = t2

        bs = jnp.max(s2, axis=(0, 1), keepdims=True)
        bt = jnp.max(t2, axis=(0, 1), keepdims=True)
        bsum = jnp.sum(wh2, axis=0, keepdims=True)

        @pl.when(r == 0)
        def _():
            ms_s[...] = bs
            mt_s[...] = bt
            sum_s[...] = bsum

        @pl.when(r > 0)
        def _():
            ms_s[...] = jnp.maximum(ms_s[...], bs)
            mt_s[...] = jnp.maximum(mt_s[...], bt)
            sum_s[...] = sum_s[...] + bsum

        @pl.when(r == nr_grid - 1)
        def _():
            ms2_ref[...] = ms_s[...]
            mt2_ref[...] = mt_s[...]
            sumwh2_ref[...] = sum_s[...]


def _att2_kernel(n, nc, bc,
                 adj_ref, s_ref, wj_ref, ms_ref, mt_ref, whs_ref,
                 sumwh_ref, wd_ref, bd_ref, z_ref, acc):
    c = pl.program_id(1)

    @pl.when(c == 0)
    def _():
        acc[...] = jnp.zeros_like(acc)

    _att_accumulate(adj_ref, s_ref, wj_ref, ms_ref, mt_ref, whs_ref, acc)

    @pl.when(c == nc - 1)
    def _():
        f = sumwh_ref.shape[1]
        accv = acc[...]
        num = accv[:, :f]
        den = accv[:, f:f + 1]
        fb = sumwh_ref[...] * (1.0 / n)
        mu = jnp.where(den > 0, num / den, fb)
        z_ref[...] = jnp.dot(mu, wd_ref[...],
                             preferred_element_type=jnp.float32) + bd_ref[...]


def kernel(x, adj, W1, a1, W2, a2, W3, a3, w2_w, w2_b):
    del W3, a3
    n, din = x.shape
    h1f = W1.shape[1]
    h2f = W2.shape[1]
    br, bc = 256, 1024
    nc = pl.cdiv(n, bc)
    npad = nc * bc
    nr = npad // br

    x_p = jnp.pad(x, ((0, npad - n), (0, 0)))

    bp = 512
    np_grid = npad // bp
    whext1, s1, t1, ms1, mt1, sumwh1 = pl.pallas_call(
        functools.partial(_proj_kernel, np_grid),
        grid=(np_grid,),
        in_specs=[
            pl.BlockSpec((bp, din), lambda r: (r, 0)),
            pl.BlockSpec((din, h1f), lambda r: (0, 0)),
            pl.BlockSpec((2 * h1f, 1), lambda r: (0, 0)),
        ],
        out_specs=[
            pl.BlockSpec((bp, h1f + 1), lambda r: (r, 0)),
            pl.BlockSpec((bp, 1), lambda r: (r, 0)),
            pl.BlockSpec((bp, 1), lambda r: (r, 0)),
            pl.BlockSpec((1, 1), lambda r: (0, 0)),
            pl.BlockSpec((1, 1), lambda r: (0, 0)),
            pl.BlockSpec((1, h1f), lambda r: (0, 0)),
        ],
        out_shape=[
            jax.ShapeDtypeStruct((npad, h1f + 1), jnp.float32),
            jax.ShapeDtypeStruct((npad, 1), jnp.float32),
            jax.ShapeDtypeStruct((npad, 1), jnp.float32),
            jax.ShapeDtypeStruct((1, 1), jnp.float32),
            jax.ShapeDtypeStruct((1, 1), jnp.float32),
            jax.ShapeDtypeStruct((1, h1f), jnp.float32),
        ],
        scratch_shapes=[
            pltpu.VMEM((1, 1), jnp.float32),
            pltpu.VMEM((1, 1), jnp.float32),
            pltpu.VMEM((1, h1f), jnp.float32),
        ],
    )(x_p, W1, a1)

    def scale_call(t_v, ms_v, mt_v, whext_v, fdim):
        brs = 512
        return pl.pallas_call(
            functools.partial(_scale_kernel, n, brs),
            grid=(npad // brs,),
            in_specs=[
                pl.BlockSpec((brs, 1), lambda r: (r, 0)),
                pl.BlockSpec((1, 1), lambda r: (0, 0)),
                pl.BlockSpec((1, 1), lambda r: (0, 0)),
                pl.BlockSpec((brs, fdim + 1), lambda r: (r, 0)),
            ],
            out_specs=[
                pl.BlockSpec((brs, fdim + 1), lambda r: (r, 0)),
                pl.BlockSpec((brs, 1), lambda r: (r, 0)),
            ],
            out_shape=[
                jax.ShapeDtypeStruct((npad, fdim + 1), jnp.float32),
                jax.ShapeDtypeStruct((npad, 1), jnp.float32),
            ],
        )(t_v, ms_v, mt_v, whext_v)

    whs1, wj1 = scale_call(t1, ms1, mt1, whext1, h1f)

    whext2, s2, t2, ms2, mt2, sumwh2 = pl.pallas_call(
        functools.partial(_att1_kernel, n, nr, nc, br, bc),
        grid=(nr, nc),
        in_specs=[
            pl.BlockSpec((br, bc), lambda r, c: (r, c)),
            pl.BlockSpec((br, 1), lambda r, c: (r, 0)),
            pl.BlockSpec((1, bc), lambda r, c: (0, c)),
            pl.BlockSpec((1, 1), lambda r, c: (0, 0)),
            pl.BlockSpec((1, 1), lambda r, c: (0, 0)),
            pl.BlockSpec((bc, h1f + 1), lambda r, c: (c, 0)),
            pl.BlockSpec((1, h1f), lambda r, c: (0, 0)),
            pl.BlockSpec((h1f, h2f), lambda r, c: (0, 0)),
            pl.BlockSpec((2 * h2f, 1), lambda r, c: (0, 0)),
        ],
        out_specs=[
            pl.BlockSpec((br, h2f + 1), lambda r, c: (r, 0)),
            pl.BlockSpec((br, 1), lambda r, c: (r, 0)),
            pl.BlockSpec((br, 1), lambda r, c: (r, 0)),
            pl.BlockSpec((1, 1), lambda r, c: (0, 0)),
            pl.BlockSpec((1, 1), lambda r, c: (0, 0)),
            pl.BlockSpec((1, h2f), lambda r, c: (0, 0)),
        ],
        out_shape=[
            jax.ShapeDtypeStruct((npad, h2f + 1), jnp.float32),
            jax.ShapeDtypeStruct((npad, 1), jnp.float32),
            jax.ShapeDtypeStruct((npad, 1), jnp.float32),
            jax.ShapeDtypeStruct((1, 1), jnp.float32),
            jax.ShapeDtypeStruct((1, 1), jnp.float32),
            jax.ShapeDtypeStruct((1, h2f), jnp.float32),
        ],
        scratch_shapes=[
            pltpu.VMEM((br, h1f + 1), jnp.float32),
            pltpu.VMEM((1, 1), jnp.float32),
            pltpu.VMEM((1, 1), jnp.float32),
            pltpu.VMEM((1, h2f), jnp.float32),
        ],
    )(adj, s1, wj1.reshape(1, npad), ms1, mt1, whs1, sumwh1, W2, a2)

    whs2, wj2 = scale_call(t2, ms2, mt2, whext2, h2f)

    z = pl.pallas_call(
        functools.partial(_att2_kernel, n, nc, bc),
        grid=(nr, nc),
        in_specs=[
            pl.BlockSpec((br, bc), lambda r, c: (r, c)),
            pl.BlockSpec((br, 1), lambda r, c: (r, 0)),
            pl.BlockSpec((1, bc), lambda r, c: (0, c)),
            pl.BlockSpec((1, 1), lambda r, c: (0, 0)),
            pl.BlockSpec((1, 1), lambda r, c: (0, 0)),
            pl.BlockSpec((bc, h2f + 1), lambda r, c: (c, 0)),
            pl.BlockSpec((1, h2f), lambda r, c: (0, 0)),
            pl.BlockSpec((h2f, din), lambda r, c: (0, 0)),
            pl.BlockSpec((1, din), lambda r, c: (0, 0)),
        ],
        out_specs=pl.BlockSpec((br, din), lambda r, c: (r, 0)),
        out_shape=jax.ShapeDtypeStruct((n, din), jnp.float32),
        scratch_shapes=[
            pltpu.VMEM((br, h2f + 1), jnp.float32),
        ],
    )(adj, s2, wj2.reshape(1, npad), ms2, mt2, whs2, sumwh2, w2_w,
      w2_b.reshape(1, din))

    return z


# speedup vs baseline: 1.1338x; 1.0410x over previous
import functools

import jax
import jax.numpy as jnp
from jax.experimental import pallas as pl
from jax.experimental.pallas import tpu as pltpu


def _proj_kernel(nr_grid, h_ref, w_ref, a_ref,
                 whext_ref, s_ref, t_ref, ms_ref, mt_ref, sumwh_ref,
                 ms_s, mt_s, sum_s):
    r = pl.program_id(0)
    f = w_ref.shape[1]
    wh = jnp.dot(h_ref[...], w_ref[...], preferred_element_type=jnp.float32)
    s = jnp.dot(wh, a_ref[:f, :], preferred_element_type=jnp.float32)
    t = jnp.dot(wh, a_ref[f:, :], preferred_element_type=jnp.float32)
    whext_ref[...] = jnp.concatenate(
        [wh, jnp.ones((wh.shape[0], 1), jnp.float32)], axis=1)
    s_ref[...] = s
    t_ref[...] = t

    bs = jnp.max(s, axis=(0, 1), keepdims=True)
    bt = jnp.max(t, axis=(0, 1), keepdims=True)
    bsum = jnp.sum(wh, axis=0, keepdims=True)

    @pl.when(r == 0)
    def _():
        ms_s[...] = bs
        mt_s[...] = bt
        sum_s[...] = bsum

    @pl.when(r > 0)
    def _():
        ms_s[...] = jnp.maximum(ms_s[...], bs)
        mt_s[...] = jnp.maximum(mt_s[...], bt)
        sum_s[...] = sum_s[...] + bsum

    @pl.when(r == nr_grid - 1)
    def _():
        ms_ref[...] = ms_s[...]
        mt_ref[...] = mt_s[...]
        sumwh_ref[...] = sum_s[...]


def _scale_kernel(n, brs, t_ref, ms_ref, mt_ref, whext_ref,
                  whs_ref, wj_ref):
    r = pl.program_id(0)
    mtv = mt_ref[...]
    del ms_ref
    t = t_ref[...]
    rowid = r * brs + jax.lax.broadcasted_iota(jnp.int32, t.shape, 0)
    v2 = jnp.where(rowid < n, jnp.exp(0.2 * (t - mtv)), 0.0)
    whs_ref[...] = whext_ref[...] * v2
    wj_ref[...] = jnp.exp(0.8 * (t - mtv))


def _att_accumulate(adj_ref, s_ref, wj_ref, ms_ref, mt_ref, whs_ref, acc):
    msv = ms_ref[...]
    mtv = mt_ref[...]
    cm = msv + mtv
    cmax = jnp.where(cm > 0, cm, 0.2 * cm)
    s = s_ref[...]
    wj = wj_ref[...]
    u1 = jnp.exp(s + mtv - cmax)
    u2 = jnp.exp(0.2 * (s + mtv) - cmax)
    p = jnp.maximum(u1 * wj, u2)
    p = jnp.where(adj_ref[...].astype(jnp.int32) > 0, p, 0.0)
    acc[...] += jnp.dot(p, whs_ref[...], preferred_element_type=jnp.float32)


def _att1_kernel(n, nr_grid, nc, br, bc,
                 adj_ref, s_ref, wj_ref, ms_ref, mt_ref, whs_ref,
                 sumwh_ref, w2_ref, a2_ref,
                 whext2_ref, s2_ref, t2_ref, ms2_ref, mt2_ref, sumwh2_ref,
                 adjb_ref, acc, ms_s, mt_s, sum_s):
    c = pl.program_id(1)

    @pl.when(c == 0)
    def _():
        acc[...] = jnp.zeros_like(acc)

    adjb_ref[...] = adj_ref[...].astype(jnp.int8)
    _att_accumulate(adj_ref, s_ref, wj_ref, ms_ref, mt_ref, whs_ref, acc)

    @pl.when(c == nc - 1)
    def _():
        r = pl.program_id(0)
        f = sumwh_ref.shape[1]
        accv = acc[...]
        num = accv[:, :f]
        den = accv[:, f:f + 1]
        fb = sumwh_ref[...] * (1.0 / n)
        h1 = jnp.where(den > 0, num / den, fb)
        h1 = jnp.maximum(h1, 0.0)
        rowid = r * br + jax.lax.broadcasted_iota(jnp.int32, h1.shape, 0)
        h1 = jnp.where(rowid < n, h1, 0.0)

        f2 = w2_ref.shape[1]
        wh2 = jnp.dot(h1, w2_ref[...], preferred_element_type=jnp.float32)
        s2 = jnp.dot(wh2, a2_ref[:f2, :], preferred_element_type=jnp.float32)
        t2 = jnp.dot(wh2, a2_ref[f2:, :], preferred_element_type=jnp.float32)
        whext2_ref[...] = jnp.concatenate(
            [wh2, jnp.ones((wh2.shape[0], 1), jnp.float32)], axis=1)
        s2_ref[...] = s2
        t2_ref[...] = t2

        bs = jnp.max(s2, axis=(0, 1), keepdims=True)
        bt = jnp.max(t2, axis=(0, 1), keepdims=True)
        bsum = jnp.sum(wh2, axis=0, keepdims=True)

        @pl.when(r == 0)
        def _():
            ms_s[...] = bs
            mt_s[...] = bt
            sum_s[...] = bsum

        @pl.when(r > 0)
        def _():
            ms_s[...] = jnp.maximum(ms_s[...], bs)
            mt_s[...] = jnp.maximum(mt_s[...], bt)
            sum_s[...] = sum_s[...] + bsum

        @pl.when(r == nr_grid - 1)
        def _():
            ms2_ref[...] = ms_s[...]
            mt2_ref[...] = mt_s[...]
            sumwh2_ref[...] = sum_s[...]


def _att2_kernel(n, nc, bc,
                 adj_ref, s_ref, wj_ref, ms_ref, mt_ref, whs_ref,
                 sumwh_ref, wd_ref, bd_ref, z_ref, acc):
    c = pl.program_id(1)

    @pl.when(c == 0)
    def _():
        acc[...] = jnp.zeros_like(acc)

    _att_accumulate(adj_ref, s_ref, wj_ref, ms_ref, mt_ref, whs_ref, acc)

    @pl.when(c == nc - 1)
    def _():
        f = sumwh_ref.shape[1]
        accv = acc[...]
        num = accv[:, :f]
        den = accv[:, f:f + 1]
        fb = sumwh_ref[...] * (1.0 / n)
        mu = jnp.where(den > 0, num / den, fb)
        z_ref[...] = jnp.dot(mu, wd_ref[...],
                             preferred_element_type=jnp.float32) + bd_ref[...]


def kernel(x, adj, W1, a1, W2, a2, W3, a3, w2_w, w2_b):
    del W3, a3
    n, din = x.shape
    h1f = W1.shape[1]
    h2f = W2.shape[1]
    br, bc = 256, 1024
    nc = pl.cdiv(n, bc)
    npad = nc * bc
    nr = npad // br

    x_p = jnp.pad(x, ((0, npad - n), (0, 0)))

    bp = 512
    np_grid = npad // bp
    whext1, s1, t1, ms1, mt1, sumwh1 = pl.pallas_call(
        functools.partial(_proj_kernel, np_grid),
        grid=(np_grid,),
        in_specs=[
            pl.BlockSpec((bp, din), lambda r: (r, 0)),
            pl.BlockSpec((din, h1f), lambda r: (0, 0)),
            pl.BlockSpec((2 * h1f, 1), lambda r: (0, 0)),
        ],
        out_specs=[
            pl.BlockSpec((bp, h1f + 1), lambda r: (r, 0)),
            pl.BlockSpec((bp, 1), lambda r: (r, 0)),
            pl.BlockSpec((bp, 1), lambda r: (r, 0)),
            pl.BlockSpec((1, 1), lambda r: (0, 0)),
            pl.BlockSpec((1, 1), lambda r: (0, 0)),
            pl.BlockSpec((1, h1f), lambda r: (0, 0)),
        ],
        out_shape=[
            jax.ShapeDtypeStruct((npad, h1f + 1), jnp.float32),
            jax.ShapeDtypeStruct((npad, 1), jnp.float32),
            jax.ShapeDtypeStruct((npad, 1), jnp.float32),
            jax.ShapeDtypeStruct((1, 1), jnp.float32),
            jax.ShapeDtypeStruct((1, 1), jnp.float32),
            jax.ShapeDtypeStruct((1, h1f), jnp.float32),
        ],
        scratch_shapes=[
            pltpu.VMEM((1, 1), jnp.float32),
            pltpu.VMEM((1, 1), jnp.float32),
            pltpu.VMEM((1, h1f), jnp.float32),
        ],
    )(x_p, W1, a1)

    def scale_call(t_v, ms_v, mt_v, whext_v, fdim):
        brs = 512
        return pl.pallas_call(
            functools.partial(_scale_kernel, n, brs),
            grid=(npad // brs,),
            in_specs=[
                pl.BlockSpec((brs, 1), lambda r: (r, 0)),
                pl.BlockSpec((1, 1), lambda r: (0, 0)),
                pl.BlockSpec((1, 1), lambda r: (0, 0)),
                pl.BlockSpec((brs, fdim + 1), lambda r: (r, 0)),
            ],
            out_specs=[
                pl.BlockSpec((brs, fdim + 1), lambda r: (r, 0)),
                pl.BlockSpec((brs, 1), lambda r: (r, 0)),
            ],
            out_shape=[
                jax.ShapeDtypeStruct((npad, fdim + 1), jnp.float32),
                jax.ShapeDtypeStruct((npad, 1), jnp.float32),
            ],
        )(t_v, ms_v, mt_v, whext_v)

    whs1, wj1 = scale_call(t1, ms1, mt1, whext1, h1f)

    whext2, s2, t2, ms2, mt2, sumwh2, adjb = pl.pallas_call(
        functools.partial(_att1_kernel, n, nr, nc, br, bc),
        grid=(nr, nc),
        in_specs=[
            pl.BlockSpec((br, bc), lambda r, c: (r, c)),
            pl.BlockSpec((br, 1), lambda r, c: (r, 0)),
            pl.BlockSpec((1, bc), lambda r, c: (0, c)),
            pl.BlockSpec((1, 1), lambda r, c: (0, 0)),
            pl.BlockSpec((1, 1), lambda r, c: (0, 0)),
            pl.BlockSpec((bc, h1f + 1), lambda r, c: (c, 0)),
            pl.BlockSpec((1, h1f), lambda r, c: (0, 0)),
            pl.BlockSpec((h1f, h2f), lambda r, c: (0, 0)),
            pl.BlockSpec((2 * h2f, 1), lambda r, c: (0, 0)),
        ],
        out_specs=[
            pl.BlockSpec((br, h2f + 1), lambda r, c: (r, 0)),
            pl.BlockSpec((br, 1), lambda r, c: (r, 0)),
            pl.BlockSpec((br, 1), lambda r, c: (r, 0)),
            pl.BlockSpec((1, 1), lambda r, c: (0, 0)),
            pl.BlockSpec((1, 1), lambda r, c: (0, 0)),
            pl.BlockSpec((1, h2f), lambda r, c: (0, 0)),
            pl.BlockSpec((br, bc), lambda r, c: (r, c)),
        ],
        out_shape=[
            jax.ShapeDtypeStruct((npad, h2f + 1), jnp.float32),
            jax.ShapeDtypeStruct((npad, 1), jnp.float32),
            jax.ShapeDtypeStruct((npad, 1), jnp.float32),
            jax.ShapeDtypeStruct((1, 1), jnp.float32),
            jax.ShapeDtypeStruct((1, 1), jnp.float32),
            jax.ShapeDtypeStruct((1, h2f), jnp.float32),
            jax.ShapeDtypeStruct((n, n), jnp.int8),
        ],
        scratch_shapes=[
            pltpu.VMEM((br, h1f + 1), jnp.float32),
            pltpu.VMEM((1, 1), jnp.float32),
            pltpu.VMEM((1, 1), jnp.float32),
            pltpu.VMEM((1, h2f), jnp.float32),
        ],
    )(adj, s1, wj1.reshape(1, npad), ms1, mt1, whs1, sumwh1, W2, a2)

    whs2, wj2 = scale_call(t2, ms2, mt2, whext2, h2f)

    z = pl.pallas_call(
        functools.partial(_att2_kernel, n, nc, bc),
        grid=(nr, nc),
        in_specs=[
            pl.BlockSpec((br, bc), lambda r, c: (r, c)),
            pl.BlockSpec((br, 1), lambda r, c: (r, 0)),
            pl.BlockSpec((1, bc), lambda r, c: (0, c)),
            pl.BlockSpec((1, 1), lambda r, c: (0, 0)),
            pl.BlockSpec((1, 1), lambda r, c: (0, 0)),
            pl.BlockSpec((bc, h2f + 1), lambda r, c: (c, 0)),
            pl.BlockSpec((1, h2f), lambda r, c: (0, 0)),
            pl.BlockSpec((h2f, din), lambda r, c: (0, 0)),
            pl.BlockSpec((1, din), lambda r, c: (0, 0)),
        ],
        out_specs=pl.BlockSpec((br, din), lambda r, c: (r, 0)),
        out_shape=jax.ShapeDtypeStruct((n, din), jnp.float32),
        scratch_shapes=[
            pltpu.VMEM((br, h2f + 1), jnp.float32),
        ],
    )(adjb, s2, wj2.reshape(1, npad), ms2, mt2, whs2, sumwh2, w2_w,
      w2_b.reshape(1, din))

    return z


# speedup vs baseline: 1.7444x; 1.5385x over previous
import functools

import jax
import jax.numpy as jnp
from jax.experimental import pallas as pl
from jax.experimental.pallas import tpu as pltpu


def _proj_kernel(nr_grid, h_ref, w_ref, a_ref,
                 whext_ref, s_ref, t_ref, ms_ref, mt_ref, sumwh_ref,
                 ms_s, mt_s, sum_s):
    r = pl.program_id(0)
    f = w_ref.shape[1]
    wh = jnp.dot(h_ref[...], w_ref[...], preferred_element_type=jnp.float32)
    s = jnp.dot(wh, a_ref[:f, :], preferred_element_type=jnp.float32)
    t = jnp.dot(wh, a_ref[f:, :], preferred_element_type=jnp.float32)
    whext_ref[...] = jnp.concatenate(
        [wh, jnp.ones((wh.shape[0], 1), jnp.float32)], axis=1)
    s_ref[...] = s
    t_ref[...] = t

    bs = jnp.max(s, axis=(0, 1), keepdims=True)
    bt = jnp.max(t, axis=(0, 1), keepdims=True)
    bsum = jnp.sum(wh, axis=0, keepdims=True)

    @pl.when(r == 0)
    def _():
        ms_s[...] = bs
        mt_s[...] = bt
        sum_s[...] = bsum

    @pl.when(r > 0)
    def _():
        ms_s[...] = jnp.maximum(ms_s[...], bs)
        mt_s[...] = jnp.maximum(mt_s[...], bt)
        sum_s[...] = sum_s[...] + bsum

    @pl.when(r == nr_grid - 1)
    def _():
        ms_ref[...] = ms_s[...]
        mt_ref[...] = mt_s[...]
        sumwh_ref[...] = sum_s[...]


def _scale_kernel(n, brs, t_ref, ms_ref, mt_ref, whext_ref,
                  whs_ref, wj_ref):
    r = pl.program_id(0)
    mtv = mt_ref[...]
    del ms_ref
    t = t_ref[...]
    rowid = r * brs + jax.lax.broadcasted_iota(jnp.int32, t.shape, 0)
    v2 = jnp.where(rowid < n, jnp.exp(0.2 * (t - mtv)), 0.0)
    whs_ref[...] = whext_ref[...] * v2
    wj_ref[...] = jnp.exp(0.8 * (t - mtv))


def _att_accumulate(adj_ref, s_ref, wj_ref, ms_ref, mt_ref, whs_ref, acc):
    msv = ms_ref[...]
    mtv = mt_ref[...]
    cm = msv + mtv
    cmax = jnp.where(cm > 0, cm, 0.2 * cm)
    s = s_ref[...]
    wj = wj_ref[...]
    u1 = jnp.exp(s + mtv - cmax)
    u2 = jnp.exp(0.2 * (s + mtv) - cmax)
    p = jnp.maximum(u1 * wj, u2)
    p = jnp.where(adj_ref[...].astype(jnp.int32) > 0, p, 0.0)
    acc[...] += jnp.dot(p, whs_ref[...], preferred_element_type=jnp.float32)


def _att1_kernel(n, nr_grid, nc, br, bc,
                 adj_ref, s_ref, wj_ref, ms_ref, mt_ref, whs_ref,
                 sumwh_ref, w2_ref, a2_ref,
                 whext2_ref, s2_ref, t2_ref, ms2_ref, mt2_ref, sumwh2_ref,
                 adjb_ref, acc, ms_s, mt_s, sum_s):
    c = pl.program_id(1)

    @pl.when(c == 0)
    def _():
        acc[...] = jnp.zeros_like(acc)

    adjb_ref[...] = adj_ref[...].astype(jnp.int8)
    _att_accumulate(adj_ref, s_ref, wj_ref, ms_ref, mt_ref, whs_ref, acc)

    @pl.when(c == nc - 1)
    def _():
        r = pl.program_id(0)
        f = sumwh_ref.shape[1]
        accv = acc[...]
        num = accv[:, :f]
        den = accv[:, f:f + 1]
        fb = sumwh_ref[...] * (1.0 / n)
        h1 = jnp.where(den > 0, num / den, fb)
        h1 = jnp.maximum(h1, 0.0)
        rowid = r * br + jax.lax.broadcasted_iota(jnp.int32, h1.shape, 0)
        h1 = jnp.where(rowid < n, h1, 0.0)

        f2 = w2_ref.shape[1]
        wh2 = jnp.dot(h1, w2_ref[...], preferred_element_type=jnp.float32)
        s2 = jnp.dot(wh2, a2_ref[:f2, :], preferred_element_type=jnp.float32)
        t2 = jnp.dot(wh2, a2_ref[f2:, :], preferred_element_type=jnp.float32)
        whext2_ref[...] = jnp.concatenate(
            [wh2, jnp.ones((wh2.shape[0], 1), jnp.float32)], axis=1)
        s2_ref[...] = s2
        t2_ref[...] = t2

        bs = jnp.max(s2, axis=(0, 1), keepdims=True)
        bt = jnp.max(t2, axis=(0, 1), keepdims=True)
        bsum = jnp.sum(wh2, axis=0, keepdims=True)

        @pl.when(r == 0)
        def _():
            ms_s[...] = bs
            mt_s[...] = bt
            sum_s[...] = bsum

        @pl.when(r > 0)
        def _():
            ms_s[...] = jnp.maximum(ms_s[...], bs)
            mt_s[...] = jnp.maximum(mt_s[...], bt)
            sum_s[...] = sum_s[...] + bsum

        @pl.when(r == nr_grid - 1)
        def _():
            ms2_ref[...] = ms_s[...]
            mt2_ref[...] = mt_s[...]
            sumwh2_ref[...] = sum_s[...]


def _att2_kernel(n, nc, bc,
                 adj_ref, s_ref, wj_ref, ms_ref, mt_ref, whs_ref,
                 sumwh_ref, wd_ref, bd_ref, z_ref, acc):
    c = pl.program_id(1)

    @pl.when(c == 0)
    def _():
        acc[...] = jnp.zeros_like(acc)

    _att_accumulate(adj_ref, s_ref, wj_ref, ms_ref, mt_ref, whs_ref, acc)

    @pl.when(c == nc - 1)
    def _():
        f = sumwh_ref.shape[1]
        accv = acc[...]
        num = accv[:, :f]
        den = accv[:, f:f + 1]
        fb = sumwh_ref[...] * (1.0 / n)
        mu = jnp.where(den > 0, num / den, fb)
        z_ref[...] = jnp.dot(mu, wd_ref[...],
                             preferred_element_type=jnp.float32) + bd_ref[...]


def kernel(x, adj, W1, a1, W2, a2, W3, a3, w2_w, w2_b):
    del W3, a3
    n, din = x.shape
    h1f = W1.shape[1]
    h2f = W2.shape[1]
    br, bc = 512, 1024
    nc = pl.cdiv(n, bc)
    npad = nc * bc
    nr = npad // br

    x_p = jnp.pad(x, ((0, npad - n), (0, 0)))

    bp = 512
    np_grid = npad // bp
    whext1, s1, t1, ms1, mt1, sumwh1 = pl.pallas_call(
        functools.partial(_proj_kernel, np_grid),
        grid=(np_grid,),
        in_specs=[
            pl.BlockSpec((bp, din), lambda r: (r, 0)),
            pl.BlockSpec((din, h1f), lambda r: (0, 0)),
            pl.BlockSpec((2 * h1f, 1), lambda r: (0, 0)),
        ],
        out_specs=[
            pl.BlockSpec((bp, h1f + 1), lambda r: (r, 0)),
            pl.BlockSpec((bp, 1), lambda r: (r, 0)),
            pl.BlockSpec((bp, 1), lambda r: (r, 0)),
            pl.BlockSpec((1, 1), lambda r: (0, 0)),
            pl.BlockSpec((1, 1), lambda r: (0, 0)),
            pl.BlockSpec((1, h1f), lambda r: (0, 0)),
        ],
        out_shape=[
            jax.ShapeDtypeStruct((npad, h1f + 1), jnp.float32),
            jax.ShapeDtypeStruct((npad, 1), jnp.float32),
            jax.ShapeDtypeStruct((npad, 1), jnp.float32),
            jax.ShapeDtypeStruct((1, 1), jnp.float32),
            jax.ShapeDtypeStruct((1, 1), jnp.float32),
            jax.ShapeDtypeStruct((1, h1f), jnp.float32),
        ],
        scratch_shapes=[
            pltpu.VMEM((1, 1), jnp.float32),
            pltpu.VMEM((1, 1), jnp.float32),
            pltpu.VMEM((1, h1f), jnp.float32),
        ],
    )(x_p, W1, a1)

    def scale_call(t_v, ms_v, mt_v, whext_v, fdim):
        brs = 512
        return pl.pallas_call(
            functools.partial(_scale_kernel, n, brs),
            grid=(npad // brs,),
            in_specs=[
                pl.BlockSpec((brs, 1), lambda r: (r, 0)),
                pl.BlockSpec((1, 1), lambda r: (0, 0)),
                pl.BlockSpec((1, 1), lambda r: (0, 0)),
                pl.BlockSpec((brs, fdim + 1), lambda r: (r, 0)),
            ],
            out_specs=[
                pl.BlockSpec((brs, fdim + 1), lambda r: (r, 0)),
                pl.BlockSpec((brs, 1), lambda r: (r, 0)),
            ],
            out_shape=[
                jax.ShapeDtypeStruct((npad, fdim + 1), jnp.float32),
                jax.ShapeDtypeStruct((npad, 1), jnp.float32),
            ],
        )(t_v, ms_v, mt_v, whext_v)

    whs1, wj1 = scale_call(t1, ms1, mt1, whext1, h1f)

    whext2, s2, t2, ms2, mt2, sumwh2, adjb = pl.pallas_call(
        functools.partial(_att1_kernel, n, nr, nc, br, bc),
        grid=(nr, nc),
        in_specs=[
            pl.BlockSpec((br, bc), lambda r, c: (r, c)),
            pl.BlockSpec((br, 1), lambda r, c: (r, 0)),
            pl.BlockSpec((1, bc), lambda r, c: (0, c)),
            pl.BlockSpec((1, 1), lambda r, c: (0, 0)),
            pl.BlockSpec((1, 1), lambda r, c: (0, 0)),
            pl.BlockSpec((bc, h1f + 1), lambda r, c: (c, 0)),
            pl.BlockSpec((1, h1f), lambda r, c: (0, 0)),
            pl.BlockSpec((h1f, h2f), lambda r, c: (0, 0)),
            pl.BlockSpec((2 * h2f, 1), lambda r, c: (0, 0)),
        ],
        out_specs=[
            pl.BlockSpec((br, h2f + 1), lambda r, c: (r, 0)),
            pl.BlockSpec((br, 1), lambda r, c: (r, 0)),
            pl.BlockSpec((br, 1), lambda r, c: (r, 0)),
            pl.BlockSpec((1, 1), lambda r, c: (0, 0)),
            pl.BlockSpec((1, 1), lambda r, c: (0, 0)),
            pl.BlockSpec((1, h2f), lambda r, c: (0, 0)),
            pl.BlockSpec((br, bc), lambda r, c: (r, c)),
        ],
        out_shape=[
            jax.ShapeDtypeStruct((npad, h2f + 1), jnp.float32),
            jax.ShapeDtypeStruct((npad, 1), jnp.float32),
            jax.ShapeDtypeStruct((npad, 1), jnp.float32),
            jax.ShapeDtypeStruct((1, 1), jnp.float32),
            jax.ShapeDtypeStruct((1, 1), jnp.float32),
            jax.ShapeDtypeStruct((1, h2f), jnp.float32),
            jax.ShapeDtypeStruct((n, n), jnp.int8),
        ],
        scratch_shapes=[
            pltpu.VMEM((br, h1f + 1), jnp.float32),
            pltpu.VMEM((1, 1), jnp.float32),
            pltpu.VMEM((1, 1), jnp.float32),
            pltpu.VMEM((1, h2f), jnp.float32),
        ],
    )(adj, s1, wj1.reshape(1, npad), ms1, mt1, whs1, sumwh1, W2, a2)

    whs2, wj2 = scale_call(t2, ms2, mt2, whext2, h2f)

    z = pl.pallas_call(
        functools.partial(_att2_kernel, n, nc, bc),
        grid=(nr, nc),
        in_specs=[
            pl.BlockSpec((br, bc), lambda r, c: (r, c)),
            pl.BlockSpec((br, 1), lambda r, c: (r, 0)),
            pl.BlockSpec((1, bc), lambda r, c: (0, c)),
            pl.BlockSpec((1, 1), lambda r, c: (0, 0)),
            pl.BlockSpec((1, 1), lambda r, c: (0, 0)),
            pl.BlockSpec((bc, h2f + 1), lambda r, c: (c, 0)),
            pl.BlockSpec((1, h2f), lambda r, c: (0, 0)),
            pl.BlockSpec((h2f, din), lambda r, c: (0, 0)),
            pl.BlockSpec((1, din), lambda r, c: (0, 0)),
        ],
        out_specs=pl.BlockSpec((br, din), lambda r, c: (r, 0)),
        out_shape=jax.ShapeDtypeStruct((n, din), jnp.float32),
        scratch_shapes=[
            pltpu.VMEM((br, h2f + 1), jnp.float32),
        ],
    )(adjb, s2, wj2.reshape(1, npad), ms2, mt2, whs2, sumwh2, w2_w,
      w2_b.reshape(1, din))

    return z


# speedup vs baseline: 2.3429x; 1.3431x over previous
import functools

import jax
import jax.numpy as jnp
from jax.experimental import pallas as pl
from jax.experimental.pallas import tpu as pltpu


def _proj_kernel(nr_grid, h_ref, w_ref, a_ref,
                 whext_ref, s_ref, t_ref, ms_ref, mt_ref, sumwh_ref,
                 ms_s, mt_s, sum_s):
    r = pl.program_id(0)
    f = w_ref.shape[1]
    wh = jnp.dot(h_ref[...], w_ref[...], preferred_element_type=jnp.float32)
    s = jnp.dot(wh, a_ref[:f, :], preferred_element_type=jnp.float32)
    t = jnp.dot(wh, a_ref[f:, :], preferred_element_type=jnp.float32)
    whext_ref[...] = jnp.concatenate(
        [wh, jnp.ones((wh.shape[0], 1), jnp.float32)], axis=1)
    s_ref[...] = s
    t_ref[...] = t

    bs = jnp.max(s, axis=(0, 1), keepdims=True)
    bt = jnp.max(t, axis=(0, 1), keepdims=True)
    bsum = jnp.sum(wh, axis=0, keepdims=True)

    @pl.when(r == 0)
    def _():
        ms_s[...] = bs
        mt_s[...] = bt
        sum_s[...] = bsum

    @pl.when(r > 0)
    def _():
        ms_s[...] = jnp.maximum(ms_s[...], bs)
        mt_s[...] = jnp.maximum(mt_s[...], bt)
        sum_s[...] = sum_s[...] + bsum

    @pl.when(r == nr_grid - 1)
    def _():
        ms_ref[...] = ms_s[...]
        mt_ref[...] = mt_s[...]
        sumwh_ref[...] = sum_s[...]


def _scale_kernel(n, brs, t_ref, ms_ref, mt_ref, whext_ref,
                  whs_ref, wj_ref):
    r = pl.program_id(0)
    mtv = mt_ref[...]
    del ms_ref
    t = t_ref[...]
    rowid = r * brs + jax.lax.broadcasted_iota(jnp.int32, t.shape, 0)
    v2 = jnp.where(rowid < n, jnp.exp(0.2 * (t - mtv)), 0.0)
    whs_ref[...] = whext_ref[...] * v2
    wj_ref[...] = jnp.exp(0.8 * (t - mtv))


def _att_accumulate(adj_ref, s_ref, wj_ref, ms_ref, mt_ref, whs_ref, acc):
    msv = ms_ref[...]
    mtv = mt_ref[...]
    cm = msv + mtv
    cmax = jnp.where(cm > 0, cm, 0.2 * cm)
    s = s_ref[...]
    wj = wj_ref[...]
    u1 = jnp.exp(s + mtv - cmax)
    u2 = jnp.exp(0.2 * (s + mtv) - cmax)
    p = jnp.maximum(u1 * wj, u2)
    p = jnp.where(adj_ref[...].astype(jnp.int32) > 0, p, 0.0)
    acc[...] += jnp.dot(p, whs_ref[...], preferred_element_type=jnp.float32)


def _att1_kernel(n, nr_grid, nc, br, bc,
                 adj_ref, s_ref, wj_ref, ms_ref, mt_ref, whs_ref,
                 sumwh_ref, w2_ref, a2_ref,
                 whext2_ref, s2_ref, t2_ref, ms2_ref, mt2_ref, sumwh2_ref,
                 adjb_ref, acc, ms_s, mt_s, sum_s):
    c = pl.program_id(1)

    @pl.when(c == 0)
    def _():
        acc[...] = jnp.zeros_like(acc)

    adjb_ref[...] = adj_ref[...].astype(jnp.int8)
    _att_accumulate(adj_ref, s_ref, wj_ref, ms_ref, mt_ref, whs_ref, acc)

    @pl.when(c == nc - 1)
    def _():
        r = pl.program_id(0)
        f = sumwh_ref.shape[1]
        accv = acc[...]
        num = accv[:, :f]
        den = accv[:, f:f + 1]
        fb = sumwh_ref[...] * (1.0 / n)
        h1 = jnp.where(den > 0, num / den, fb)
        h1 = jnp.maximum(h1, 0.0)
        rowid = r * br + jax.lax.broadcasted_iota(jnp.int32, h1.shape, 0)
        h1 = jnp.where(rowid < n, h1, 0.0)

        f2 = w2_ref.shape[1]
        wh2 = jnp.dot(h1, w2_ref[...], preferred_element_type=jnp.float32)
        s2 = jnp.dot(wh2, a2_ref[:f2, :], preferred_element_type=jnp.float32)
        t2 = jnp.dot(wh2, a2_ref[f2:, :], preferred_element_type=jnp.float32)
        whext2_ref[...] = jnp.concatenate(
            [wh2, jnp.ones((wh2.shape[0], 1), jnp.float32)], axis=1)
        s2_ref[...] = s2
        t2_ref[...] = t2

        bs = jnp.max(s2, axis=(0, 1), keepdims=True)
        bt = jnp.max(t2, axis=(0, 1), keepdims=True)
        bsum = jnp.sum(wh2, axis=0, keepdims=True)

        @pl.when(r == 0)
        def _():
            ms_s[...] = bs
            mt_s[...] = bt
            sum_s[...] = bsum

        @pl.when(r > 0)
        def _():
            ms_s[...] = jnp.maximum(ms_s[...], bs)
            mt_s[...] = jnp.maximum(mt_s[...], bt)
            sum_s[...] = sum_s[...] + bsum

        @pl.when(r == nr_grid - 1)
        def _():
            ms2_ref[...] = ms_s[...]
            mt2_ref[...] = mt_s[...]
            sumwh2_ref[...] = sum_s[...]


def _att2_kernel(n, nc, bc,
                 adj_ref, s_ref, wj_ref, ms_ref, mt_ref, whs_ref,
                 sumwh_ref, wd_ref, bd_ref, z_ref, acc):
    c = pl.program_id(1)

    @pl.when(c == 0)
    def _():
        acc[...] = jnp.zeros_like(acc)

    _att_accumulate(adj_ref, s_ref, wj_ref, ms_ref, mt_ref, whs_ref, acc)

    @pl.when(c == nc - 1)
    def _():
        f = sumwh_ref.shape[1]
        accv = acc[...]
        num = accv[:, :f]
        den = accv[:, f:f + 1]
        fb = sumwh_ref[...] * (1.0 / n)
        mu = jnp.where(den > 0, num / den, fb)
        z_ref[...] = jnp.dot(mu, wd_ref[...],
                             preferred_element_type=jnp.float32) + bd_ref[...]


def kernel(x, adj, W1, a1, W2, a2, W3, a3, w2_w, w2_b):
    del W3, a3
    n, din = x.shape
    h1f = W1.shape[1]
    h2f = W2.shape[1]
    br, bc = 1024, 1024
    nc = pl.cdiv(n, bc)
    npad = nc * bc
    nr = npad // br

    x_p = jnp.pad(x, ((0, npad - n), (0, 0)))

    bp = 512
    np_grid = npad // bp
    whext1, s1, t1, ms1, mt1, sumwh1 = pl.pallas_call(
        functools.partial(_proj_kernel, np_grid),
        grid=(np_grid,),
        in_specs=[
            pl.BlockSpec((bp, din), lambda r: (r, 0)),
            pl.BlockSpec((din, h1f), lambda r: (0, 0)),
            pl.BlockSpec((2 * h1f, 1), lambda r: (0, 0)),
        ],
        out_specs=[
            pl.BlockSpec((bp, h1f + 1), lambda r: (r, 0)),
            pl.BlockSpec((bp, 1), lambda r: (r, 0)),
            pl.BlockSpec((bp, 1), lambda r: (r, 0)),
            pl.BlockSpec((1, 1), lambda r: (0, 0)),
            pl.BlockSpec((1, 1), lambda r: (0, 0)),
            pl.BlockSpec((1, h1f), lambda r: (0, 0)),
        ],
        out_shape=[
            jax.ShapeDtypeStruct((npad, h1f + 1), jnp.float32),
            jax.ShapeDtypeStruct((npad, 1), jnp.float32),
            jax.ShapeDtypeStruct((npad, 1), jnp.float32),
            jax.ShapeDtypeStruct((1, 1), jnp.float32),
            jax.ShapeDtypeStruct((1, 1), jnp.float32),
            jax.ShapeDtypeStruct((1, h1f), jnp.float32),
        ],
        scratch_shapes=[
            pltpu.VMEM((1, 1), jnp.float32),
            pltpu.VMEM((1, 1), jnp.float32),
            pltpu.VMEM((1, h1f), jnp.float32),
        ],
    )(x_p, W1, a1)

    def scale_call(t_v, ms_v, mt_v, whext_v, fdim):
        brs = 512
        return pl.pallas_call(
            functools.partial(_scale_kernel, n, brs),
            grid=(npad // brs,),
            in_specs=[
                pl.BlockSpec((brs, 1), lambda r: (r, 0)),
                pl.BlockSpec((1, 1), lambda r: (0, 0)),
                pl.BlockSpec((1, 1), lambda r: (0, 0)),
                pl.BlockSpec((brs, fdim + 1), lambda r: (r, 0)),
            ],
            out_specs=[
                pl.BlockSpec((brs, fdim + 1), lambda r: (r, 0)),
                pl.BlockSpec((brs, 1), lambda r: (r, 0)),
            ],
            out_shape=[
                jax.ShapeDtypeStruct((npad, fdim + 1), jnp.float32),
                jax.ShapeDtypeStruct((npad, 1), jnp.float32),
            ],
        )(t_v, ms_v, mt_v, whext_v)

    whs1, wj1 = scale_call(t1, ms1, mt1, whext1, h1f)

    whext2, s2, t2, ms2, mt2, sumwh2, adjb = pl.pallas_call(
        functools.partial(_att1_kernel, n, nr, nc, br, bc),
        grid=(nr, nc),
        in_specs=[
            pl.BlockSpec((br, bc), lambda r, c: (r, c)),
            pl.BlockSpec((br, 1), lambda r, c: (r, 0)),
            pl.BlockSpec((1, bc), lambda r, c: (0, c)),
            pl.BlockSpec((1, 1), lambda r, c: (0, 0)),
            pl.BlockSpec((1, 1), lambda r, c: (0, 0)),
            pl.BlockSpec((bc, h1f + 1), lambda r, c: (c, 0)),
            pl.BlockSpec((1, h1f), lambda r, c: (0, 0)),
            pl.BlockSpec((h1f, h2f), lambda r, c: (0, 0)),
            pl.BlockSpec((2 * h2f, 1), lambda r, c: (0, 0)),
        ],
        out_specs=[
            pl.BlockSpec((br, h2f + 1), lambda r, c: (r, 0)),
            pl.BlockSpec((br, 1), lambda r, c: (r, 0)),
            pl.BlockSpec((br, 1), lambda r, c: (r, 0)),
            pl.BlockSpec((1, 1), lambda r, c: (0, 0)),
            pl.BlockSpec((1, 1), lambda r, c: (0, 0)),
            pl.BlockSpec((1, h2f), lambda r, c: (0, 0)),
            pl.BlockSpec((br, bc), lambda r, c: (r, c)),
        ],
        out_shape=[
            jax.ShapeDtypeStruct((npad, h2f + 1), jnp.float32),
            jax.ShapeDtypeStruct((npad, 1), jnp.float32),
            jax.ShapeDtypeStruct((npad, 1), jnp.float32),
            jax.ShapeDtypeStruct((1, 1), jnp.float32),
            jax.ShapeDtypeStruct((1, 1), jnp.float32),
            jax.ShapeDtypeStruct((1, h2f), jnp.float32),
            jax.ShapeDtypeStruct((n, n), jnp.int8),
        ],
        scratch_shapes=[
            pltpu.VMEM((br, h1f + 1), jnp.float32),
            pltpu.VMEM((1, 1), jnp.float32),
            pltpu.VMEM((1, 1), jnp.float32),
            pltpu.VMEM((1, h2f), jnp.float32),
        ],
    )(adj, s1, wj1.reshape(1, npad), ms1, mt1, whs1, sumwh1, W2, a2)

    whs2, wj2 = scale_call(t2, ms2, mt2, whext2, h2f)

    z = pl.pallas_call(
        functools.partial(_att2_kernel, n, nc, bc),
        grid=(nr, nc),
        in_specs=[
            pl.BlockSpec((br, bc), lambda r, c: (r, c)),
            pl.BlockSpec((br, 1), lambda r, c: (r, 0)),
            pl.BlockSpec((1, bc), lambda r, c: (0, c)),
            pl.BlockSpec((1, 1), lambda r, c: (0, 0)),
            pl.BlockSpec((1, 1), lambda r, c: (0, 0)),
            pl.BlockSpec((bc, h2f + 1), lambda r, c: (c, 0)),
            pl.BlockSpec((1, h2f), lambda r, c: (0, 0)),
            pl.BlockSpec((h2f, din), lambda r, c: (0, 0)),
            pl.BlockSpec((1, din), lambda r, c: (0, 0)),
        ],
        out_specs=pl.BlockSpec((br, din), lambda r, c: (r, 0)),
        out_shape=jax.ShapeDtypeStruct((n, din), jnp.float32),
        scratch_shapes=[
            pltpu.VMEM((br, h2f + 1), jnp.float32),
        ],
    )(adjb, s2, wj2.reshape(1, npad), ms2, mt2, whs2, sumwh2, w2_w,
      w2_b.reshape(1, din))

    return z


# speedup vs baseline: 2.7691x; 1.1819x over previous
import functools

import jax
import jax.numpy as jnp
from jax.experimental import pallas as pl
from jax.experimental.pallas import tpu as pltpu


def _proj_kernel(nr_grid, h_ref, w_ref, a_ref,
                 whext_ref, s_ref, t_ref, ms_ref, mt_ref, sumwh_ref,
                 ms_s, mt_s, sum_s):
    r = pl.program_id(0)
    f = w_ref.shape[1]
    wh = jnp.dot(h_ref[...], w_ref[...], preferred_element_type=jnp.float32)
    s = jnp.dot(wh, a_ref[:f, :], preferred_element_type=jnp.float32)
    t = jnp.dot(wh, a_ref[f:, :], preferred_element_type=jnp.float32)
    whext_ref[...] = jnp.concatenate(
        [wh, jnp.ones((wh.shape[0], 1), jnp.float32)], axis=1)
    s_ref[...] = s
    t_ref[...] = t

    bs = jnp.max(s, axis=(0, 1), keepdims=True)
    bt = jnp.max(t, axis=(0, 1), keepdims=True)
    bsum = jnp.sum(wh, axis=0, keepdims=True)

    @pl.when(r == 0)
    def _():
        ms_s[...] = bs
        mt_s[...] = bt
        sum_s[...] = bsum

    @pl.when(r > 0)
    def _():
        ms_s[...] = jnp.maximum(ms_s[...], bs)
        mt_s[...] = jnp.maximum(mt_s[...], bt)
        sum_s[...] = sum_s[...] + bsum

    @pl.when(r == nr_grid - 1)
    def _():
        ms_ref[...] = ms_s[...]
        mt_ref[...] = mt_s[...]
        sumwh_ref[...] = sum_s[...]


def _scale_kernel(n, brs, t_ref, ms_ref, mt_ref, whext_ref,
                  whs_ref, wj_ref):
    r = pl.program_id(0)
    mtv = mt_ref[...]
    del ms_ref
    t = t_ref[...]
    rowid = r * brs + jax.lax.broadcasted_iota(jnp.int32, t.shape, 0)
    v2 = jnp.where(rowid < n, jnp.exp(0.2 * (t - mtv)), 0.0)
    whs_ref[...] = whext_ref[...] * v2
    wj_ref[...] = jnp.exp(0.8 * (t - mtv))


def _att_accumulate(adj_ref, s_ref, wj_ref, ms_ref, mt_ref, whs_ref, acc):
    msv = ms_ref[...]
    mtv = mt_ref[...]
    cm = msv + mtv
    cmax = jnp.where(cm > 0, cm, 0.2 * cm)
    s = s_ref[...]
    wj = wj_ref[...]
    u1 = jnp.exp(s + mtv - cmax)
    u2 = jnp.exp(0.2 * (s + mtv) - cmax)
    p = jnp.maximum(u1 * wj, u2)
    p = jnp.where(adj_ref[...].astype(jnp.int32) > 0, p, 0.0)
    acc[...] += jnp.dot(p, whs_ref[...], preferred_element_type=jnp.float32)


def _att1_kernel(n, nr_grid, nc, br, bc,
                 adj_ref, s_ref, wj_ref, ms_ref, mt_ref, whs_ref,
                 sumwh_ref, w2_ref, a2_ref,
                 whext2_ref, s2_ref, t2_ref, ms2_ref, mt2_ref, sumwh2_ref,
                 adjb_ref, acc, ms_s, mt_s, sum_s):
    c = pl.program_id(1)

    @pl.when(c == 0)
    def _():
        acc[...] = jnp.zeros_like(acc)

    adjb_ref[...] = adj_ref[...].astype(jnp.int8)
    _att_accumulate(adj_ref, s_ref, wj_ref, ms_ref, mt_ref, whs_ref, acc)

    @pl.when(c == nc - 1)
    def _():
        r = pl.program_id(0)
        f = sumwh_ref.shape[1]
        accv = acc[...]
        num = accv[:, :f]
        den = accv[:, f:f + 1]
        fb = sumwh_ref[...] * (1.0 / n)
        h1 = jnp.where(den > 0, num / den, fb)
        h1 = jnp.maximum(h1, 0.0)
        rowid = r * br + jax.lax.broadcasted_iota(jnp.int32, h1.shape, 0)
        h1 = jnp.where(rowid < n, h1, 0.0)

        f2 = w2_ref.shape[1]
        wh2 = jnp.dot(h1, w2_ref[...], preferred_element_type=jnp.float32)
        s2 = jnp.dot(wh2, a2_ref[:f2, :], preferred_element_type=jnp.float32)
        t2 = jnp.dot(wh2, a2_ref[f2:, :], preferred_element_type=jnp.float32)
        whext2_ref[...] = jnp.concatenate(
            [wh2, jnp.ones((wh2.shape[0], 1), jnp.float32)], axis=1)
        s2_ref[...] = s2
        t2_ref[...] = t2

        bs = jnp.max(s2, axis=(0, 1), keepdims=True)
        bt = jnp.max(t2, axis=(0, 1), keepdims=True)
        bsum = jnp.sum(wh2, axis=0, keepdims=True)

        @pl.when(r == 0)
        def _():
            ms_s[...] = bs
            mt_s[...] = bt
            sum_s[...] = bsum

        @pl.when(r > 0)
        def _():
            ms_s[...] = jnp.maximum(ms_s[...], bs)
            mt_s[...] = jnp.maximum(mt_s[...], bt)
            sum_s[...] = sum_s[...] + bsum

        @pl.when(r == nr_grid - 1)
        def _():
            ms2_ref[...] = ms_s[...]
            mt2_ref[...] = mt_s[...]
            sumwh2_ref[...] = sum_s[...]


def _att2_kernel(n, nc, bc,
                 adj_ref, s_ref, wj_ref, ms_ref, mt_ref, whs_ref,
                 sumwh_ref, wd_ref, bd_ref, z_ref, acc):
    c = pl.program_id(1)

    @pl.when(c == 0)
    def _():
        acc[...] = jnp.zeros_like(acc)

    _att_accumulate(adj_ref, s_ref, wj_ref, ms_ref, mt_ref, whs_ref, acc)

    @pl.when(c == nc - 1)
    def _():
        f = sumwh_ref.shape[1]
        accv = acc[...]
        num = accv[:, :f]
        den = accv[:, f:f + 1]
        fb = sumwh_ref[...] * (1.0 / n)
        mu = jnp.where(den > 0, num / den, fb)
        z_ref[...] = jnp.dot(mu, wd_ref[...],
                             preferred_element_type=jnp.float32) + bd_ref[...]


def kernel(x, adj, W1, a1, W2, a2, W3, a3, w2_w, w2_b):
    del W3, a3
    n, din = x.shape
    h1f = W1.shape[1]
    h2f = W2.shape[1]
    br, bc = 2048, 1024
    nc = pl.cdiv(n, bc)
    npad = nc * bc
    nr = npad // br

    x_p = jnp.pad(x, ((0, npad - n), (0, 0)))

    bp = 512
    np_grid = npad // bp
    whext1, s1, t1, ms1, mt1, sumwh1 = pl.pallas_call(
        functools.partial(_proj_kernel, np_grid),
        grid=(np_grid,),
        in_specs=[
            pl.BlockSpec((bp, din), lambda r: (r, 0)),
            pl.BlockSpec((din, h1f), lambda r: (0, 0)),
            pl.BlockSpec((2 * h1f, 1), lambda r: (0, 0)),
        ],
        out_specs=[
            pl.BlockSpec((bp, h1f + 1), lambda r: (r, 0)),
            pl.BlockSpec((bp, 1), lambda r: (r, 0)),
            pl.BlockSpec((bp, 1), lambda r: (r, 0)),
            pl.BlockSpec((1, 1), lambda r: (0, 0)),
            pl.BlockSpec((1, 1), lambda r: (0, 0)),
            pl.BlockSpec((1, h1f), lambda r: (0, 0)),
        ],
        out_shape=[
            jax.ShapeDtypeStruct((npad, h1f + 1), jnp.float32),
            jax.ShapeDtypeStruct((npad, 1), jnp.float32),
            jax.ShapeDtypeStruct((npad, 1), jnp.float32),
            jax.ShapeDtypeStruct((1, 1), jnp.float32),
            jax.ShapeDtypeStruct((1, 1), jnp.float32),
            jax.ShapeDtypeStruct((1, h1f), jnp.float32),
        ],
        scratch_shapes=[
            pltpu.VMEM((1, 1), jnp.float32),
            pltpu.VMEM((1, 1), jnp.float32),
            pltpu.VMEM((1, h1f), jnp.float32),
        ],
    )(x_p, W1, a1)

    def scale_call(t_v, ms_v, mt_v, whext_v, fdim):
        brs = 512
        return pl.pallas_call(
            functools.partial(_scale_kernel, n, brs),
            grid=(npad // brs,),
            in_specs=[
                pl.BlockSpec((brs, 1), lambda r: (r, 0)),
                pl.BlockSpec((1, 1), lambda r: (0, 0)),
                pl.BlockSpec((1, 1), lambda r: (0, 0)),
                pl.BlockSpec((brs, fdim + 1), lambda r: (r, 0)),
            ],
            out_specs=[
                pl.BlockSpec((brs, fdim + 1), lambda r: (r, 0)),
                pl.BlockSpec((brs, 1), lambda r: (r, 0)),
            ],
            out_shape=[
                jax.ShapeDtypeStruct((npad, fdim + 1), jnp.float32),
                jax.ShapeDtypeStruct((npad, 1), jnp.float32),
            ],
        )(t_v, ms_v, mt_v, whext_v)

    whs1, wj1 = scale_call(t1, ms1, mt1, whext1, h1f)

    whext2, s2, t2, ms2, mt2, sumwh2, adjb = pl.pallas_call(
        functools.partial(_att1_kernel, n, nr, nc, br, bc),
        grid=(nr, nc),
        in_specs=[
            pl.BlockSpec((br, bc), lambda r, c: (r, c)),
            pl.BlockSpec((br, 1), lambda r, c: (r, 0)),
            pl.BlockSpec((1, bc), lambda r, c: (0, c)),
            pl.BlockSpec((1, 1), lambda r, c: (0, 0)),
            pl.BlockSpec((1, 1), lambda r, c: (0, 0)),
            pl.BlockSpec((bc, h1f + 1), lambda r, c: (c, 0)),
            pl.BlockSpec((1, h1f), lambda r, c: (0, 0)),
            pl.BlockSpec((h1f, h2f), lambda r, c: (0, 0)),
            pl.BlockSpec((2 * h2f, 1), lambda r, c: (0, 0)),
        ],
        out_specs=[
            pl.BlockSpec((br, h2f + 1), lambda r, c: (r, 0)),
            pl.BlockSpec((br, 1), lambda r, c: (r, 0)),
            pl.BlockSpec((br, 1), lambda r, c: (r, 0)),
            pl.BlockSpec((1, 1), lambda r, c: (0, 0)),
            pl.BlockSpec((1, 1), lambda r, c: (0, 0)),
            pl.BlockSpec((1, h2f), lambda r, c: (0, 0)),
            pl.BlockSpec((br, bc), lambda r, c: (r, c)),
        ],
        out_shape=[
            jax.ShapeDtypeStruct((npad, h2f + 1), jnp.float32),
            jax.ShapeDtypeStruct((npad, 1), jnp.float32),
            jax.ShapeDtypeStruct((npad, 1), jnp.float32),
            jax.ShapeDtypeStruct((1, 1), jnp.float32),
            jax.ShapeDtypeStruct((1, 1), jnp.float32),
            jax.ShapeDtypeStruct((1, h2f), jnp.float32),
            jax.ShapeDtypeStruct((n, n), jnp.int8),
        ],
        scratch_shapes=[
            pltpu.VMEM((br, h1f + 1), jnp.float32),
            pltpu.VMEM((1, 1), jnp.float32),
            pltpu.VMEM((1, 1), jnp.float32),
            pltpu.VMEM((1, h2f), jnp.float32),
        ],
    )(adj, s1, wj1.reshape(1, npad), ms1, mt1, whs1, sumwh1, W2, a2)

    whs2, wj2 = scale_call(t2, ms2, mt2, whext2, h2f)

    z = pl.pallas_call(
        functools.partial(_att2_kernel, n, nc, bc),
        grid=(nr, nc),
        in_specs=[
            pl.BlockSpec((br, bc), lambda r, c: (r, c)),
            pl.BlockSpec((br, 1), lambda r, c: (r, 0)),
            pl.BlockSpec((1, bc), lambda r, c: (0, c)),
            pl.BlockSpec((1, 1), lambda r, c: (0, 0)),
            pl.BlockSpec((1, 1), lambda r, c: (0, 0)),
            pl.BlockSpec((bc, h2f + 1), lambda r, c: (c, 0)),
            pl.BlockSpec((1, h2f), lambda r, c: (0, 0)),
            pl.BlockSpec((h2f, din), lambda r, c: (0, 0)),
            pl.BlockSpec((1, din), lambda r, c: (0, 0)),
        ],
        out_specs=pl.BlockSpec((br, din), lambda r, c: (r, 0)),
        out_shape=jax.ShapeDtypeStruct((n, din), jnp.float32),
        scratch_shapes=[
            pltpu.VMEM((br, h2f + 1), jnp.float32),
        ],
    )(adjb, s2, wj2.reshape(1, npad), ms2, mt2, whs2, sumwh2, w2_w,
      w2_b.reshape(1, din))

    return z


# speedup vs baseline: 2.8373x; 1.0246x over previous
import functools

import jax
import jax.numpy as jnp
from jax.experimental import pallas as pl
from jax.experimental.pallas import tpu as pltpu


def _proj_kernel(nr_grid, h_ref, w_ref, a_ref,
                 whext_ref, s_ref, t_ref, ms_ref, mt_ref, sumwh_ref,
                 ms_s, mt_s, sum_s):
    r = pl.program_id(0)
    f = w_ref.shape[1]
    wh = jnp.dot(h_ref[...], w_ref[...], preferred_element_type=jnp.float32)
    s = jnp.dot(wh, a_ref[:f, :], preferred_element_type=jnp.float32)
    t = jnp.dot(wh, a_ref[f:, :], preferred_element_type=jnp.float32)
    whext_ref[...] = jnp.concatenate(
        [wh, jnp.ones((wh.shape[0], 1), jnp.float32)], axis=1)
    s_ref[...] = s
    t_ref[...] = t

    bs = jnp.max(s, axis=(0, 1), keepdims=True)
    bt = jnp.max(t, axis=(0, 1), keepdims=True)
    bsum = jnp.sum(wh, axis=0, keepdims=True)

    @pl.when(r == 0)
    def _():
        ms_s[...] = bs
        mt_s[...] = bt
        sum_s[...] = bsum

    @pl.when(r > 0)
    def _():
        ms_s[...] = jnp.maximum(ms_s[...], bs)
        mt_s[...] = jnp.maximum(mt_s[...], bt)
        sum_s[...] = sum_s[...] + bsum

    @pl.when(r == nr_grid - 1)
    def _():
        ms_ref[...] = ms_s[...]
        mt_ref[...] = mt_s[...]
        sumwh_ref[...] = sum_s[...]


def _scale_kernel(n, brs, t_ref, ms_ref, mt_ref, whext_ref,
                  whs_ref, wj_ref):
    r = pl.program_id(0)
    mtv = mt_ref[...]
    del ms_ref
    t = t_ref[...]
    rowid = r * brs + jax.lax.broadcasted_iota(jnp.int32, t.shape, 0)
    v2 = jnp.where(rowid < n, jnp.exp(0.2 * (t - mtv)), 0.0)
    whs_ref[...] = whext_ref[...] * v2
    wj_ref[...] = jnp.exp(0.8 * (t - mtv))


def _att_accumulate(adj_ref, s_ref, wj_ref, ms_ref, mt_ref, whs_ref, acc):
    msv = ms_ref[...]
    mtv = mt_ref[...]
    cm = msv + mtv
    cmax = jnp.where(cm > 0, cm, 0.2 * cm)
    s = s_ref[...]
    wj = wj_ref[...]
    u1 = jnp.exp(s + mtv - cmax)
    u2 = jnp.exp(0.2 * (s + mtv) - cmax)
    p = jnp.maximum(u1 * wj, u2)
    p = jnp.where(adj_ref[...].astype(jnp.int32) > 0, p, 0.0)
    acc[...] += jnp.dot(p, whs_ref[...], preferred_element_type=jnp.float32)


def _att1_kernel(n, nr_grid, nc, br, bc,
                 adj_ref, s_ref, wj_ref, ms_ref, mt_ref, whs_ref,
                 sumwh_ref, w2_ref, a2_ref,
                 whext2_ref, s2_ref, t2_ref, ms2_ref, mt2_ref, sumwh2_ref,
                 adjb_ref, acc, ms_s, mt_s, sum_s):
    c = pl.program_id(1)

    @pl.when(c == 0)
    def _():
        acc[...] = jnp.zeros_like(acc)

    adjb_ref[...] = adj_ref[...].astype(jnp.int8)
    _att_accumulate(adj_ref, s_ref, wj_ref, ms_ref, mt_ref, whs_ref, acc)

    @pl.when(c == nc - 1)
    def _():
        r = pl.program_id(0)
        f = sumwh_ref.shape[1]
        accv = acc[...]
        num = accv[:, :f]
        den = accv[:, f:f + 1]
        fb = sumwh_ref[...] * (1.0 / n)
        h1 = jnp.where(den > 0, num / den, fb)
        h1 = jnp.maximum(h1, 0.0)
        rowid = r * br + jax.lax.broadcasted_iota(jnp.int32, h1.shape, 0)
        h1 = jnp.where(rowid < n, h1, 0.0)

        f2 = w2_ref.shape[1]
        wh2 = jnp.dot(h1, w2_ref[...], preferred_element_type=jnp.float32)
        s2 = jnp.dot(wh2, a2_ref[:f2, :], preferred_element_type=jnp.float32)
        t2 = jnp.dot(wh2, a2_ref[f2:, :], preferred_element_type=jnp.float32)
        whext2_ref[...] = jnp.concatenate(
            [wh2, jnp.ones((wh2.shape[0], 1), jnp.float32)], axis=1)
        s2_ref[...] = s2
        t2_ref[...] = t2

        bs = jnp.max(s2, axis=(0, 1), keepdims=True)
        bt = jnp.max(t2, axis=(0, 1), keepdims=True)
        bsum = jnp.sum(wh2, axis=0, keepdims=True)

        @pl.when(r == 0)
        def _():
            ms_s[...] = bs
            mt_s[...] = bt
            sum_s[...] = bsum

        @pl.when(r > 0)
        def _():
            ms_s[...] = jnp.maximum(ms_s[...], bs)
            mt_s[...] = jnp.maximum(mt_s[...], bt)
            sum_s[...] = sum_s[...] + bsum

        @pl.when(r == nr_grid - 1)
        def _():
            ms2_ref[...] = ms_s[...]
            mt2_ref[...] = mt_s[...]
            sumwh2_ref[...] = sum_s[...]


def _att2_kernel(n, nc, bc,
                 adj_ref, s_ref, wj_ref, ms_ref, mt_ref, whs_ref,
                 sumwh_ref, wd_ref, bd_ref, z_ref, acc):
    c = pl.program_id(1)

    @pl.when(c == 0)
    def _():
        acc[...] = jnp.zeros_like(acc)

    _att_accumulate(adj_ref, s_ref, wj_ref, ms_ref, mt_ref, whs_ref, acc)

    @pl.when(c == nc - 1)
    def _():
        f = sumwh_ref.shape[1]
        accv = acc[...]
        num = accv[:, :f]
        den = accv[:, f:f + 1]
        fb = sumwh_ref[...] * (1.0 / n)
        mu = jnp.where(den > 0, num / den, fb)
        z_ref[...] = jnp.dot(mu, wd_ref[...],
                             preferred_element_type=jnp.float32) + bd_ref[...]


def kernel(x, adj, W1, a1, W2, a2, W3, a3, w2_w, w2_b):
    del W3, a3
    n, din = x.shape
    h1f = W1.shape[1]
    h2f = W2.shape[1]
    br, bc = 2560, 1024
    nc = pl.cdiv(n, bc)
    npad = nc * bc
    nr = npad // br

    x_p = jnp.pad(x, ((0, npad - n), (0, 0)))

    bp = 512
    np_grid = npad // bp
    whext1, s1, t1, ms1, mt1, sumwh1 = pl.pallas_call(
        functools.partial(_proj_kernel, np_grid),
        grid=(np_grid,),
        in_specs=[
            pl.BlockSpec((bp, din), lambda r: (r, 0)),
            pl.BlockSpec((din, h1f), lambda r: (0, 0)),
            pl.BlockSpec((2 * h1f, 1), lambda r: (0, 0)),
        ],
        out_specs=[
            pl.BlockSpec((bp, h1f + 1), lambda r: (r, 0)),
            pl.BlockSpec((bp, 1), lambda r: (r, 0)),
            pl.BlockSpec((bp, 1), lambda r: (r, 0)),
            pl.BlockSpec((1, 1), lambda r: (0, 0)),
            pl.BlockSpec((1, 1), lambda r: (0, 0)),
            pl.BlockSpec((1, h1f), lambda r: (0, 0)),
        ],
        out_shape=[
            jax.ShapeDtypeStruct((npad, h1f + 1), jnp.float32),
            jax.ShapeDtypeStruct((npad, 1), jnp.float32),
            jax.ShapeDtypeStruct((npad, 1), jnp.float32),
            jax.ShapeDtypeStruct((1, 1), jnp.float32),
            jax.ShapeDtypeStruct((1, 1), jnp.float32),
            jax.ShapeDtypeStruct((1, h1f), jnp.float32),
        ],
        scratch_shapes=[
            pltpu.VMEM((1, 1), jnp.float32),
            pltpu.VMEM((1, 1), jnp.float32),
            pltpu.VMEM((1, h1f), jnp.float32),
        ],
    )(x_p, W1, a1)

    def scale_call(t_v, ms_v, mt_v, whext_v, fdim):
        brs = 512
        return pl.pallas_call(
            functools.partial(_scale_kernel, n, brs),
            grid=(npad // brs,),
            in_specs=[
                pl.BlockSpec((brs, 1), lambda r: (r, 0)),
                pl.BlockSpec((1, 1), lambda r: (0, 0)),
                pl.BlockSpec((1, 1), lambda r: (0, 0)),
                pl.BlockSpec((brs, fdim + 1), lambda r: (r, 0)),
            ],
            out_specs=[
                pl.BlockSpec((brs, fdim + 1), lambda r: (r, 0)),
                pl.BlockSpec((brs, 1), lambda r: (r, 0)),
            ],
            out_shape=[
                jax.ShapeDtypeStruct((npad, fdim + 1), jnp.float32),
                jax.ShapeDtypeStruct((npad, 1), jnp.float32),
            ],
        )(t_v, ms_v, mt_v, whext_v)

    whs1, wj1 = scale_call(t1, ms1, mt1, whext1, h1f)

    whext2, s2, t2, ms2, mt2, sumwh2, adjb = pl.pallas_call(
        functools.partial(_att1_kernel, n, nr, nc, br, bc),
        grid=(nr, nc),
        in_specs=[
            pl.BlockSpec((br, bc), lambda r, c: (r, c)),
            pl.BlockSpec((br, 1), lambda r, c: (r, 0)),
            pl.BlockSpec((1, bc), lambda r, c: (0, c)),
            pl.BlockSpec((1, 1), lambda r, c: (0, 0)),
            pl.BlockSpec((1, 1), lambda r, c: (0, 0)),
            pl.BlockSpec((bc, h1f + 1), lambda r, c: (c, 0)),
            pl.BlockSpec((1, h1f), lambda r, c: (0, 0)),
            pl.BlockSpec((h1f, h2f), lambda r, c: (0, 0)),
            pl.BlockSpec((2 * h2f, 1), lambda r, c: (0, 0)),
        ],
        out_specs=[
            pl.BlockSpec((br, h2f + 1), lambda r, c: (r, 0)),
            pl.BlockSpec((br, 1), lambda r, c: (r, 0)),
            pl.BlockSpec((br, 1), lambda r, c: (r, 0)),
            pl.BlockSpec((1, 1), lambda r, c: (0, 0)),
            pl.BlockSpec((1, 1), lambda r, c: (0, 0)),
            pl.BlockSpec((1, h2f), lambda r, c: (0, 0)),
            pl.BlockSpec((br, bc), lambda r, c: (r, c)),
        ],
        out_shape=[
            jax.ShapeDtypeStruct((npad, h2f + 1), jnp.float32),
            jax.ShapeDtypeStruct((npad, 1), jnp.float32),
            jax.ShapeDtypeStruct((npad, 1), jnp.float32),
            jax.ShapeDtypeStruct((1, 1), jnp.float32),
            jax.ShapeDtypeStruct((1, 1), jnp.float32),
            jax.ShapeDtypeStruct((1, h2f), jnp.float32),
            jax.ShapeDtypeStruct((n, n), jnp.int8),
        ],
        scratch_shapes=[
            pltpu.VMEM((br, h1f + 1), jnp.float32),
            pltpu.VMEM((1, 1), jnp.float32),
            pltpu.VMEM((1, 1), jnp.float32),
            pltpu.VMEM((1, h2f), jnp.float32),
        ],
    )(adj, s1, wj1.reshape(1, npad), ms1, mt1, whs1, sumwh1, W2, a2)

    whs2, wj2 = scale_call(t2, ms2, mt2, whext2, h2f)

    z = pl.pallas_call(
        functools.partial(_att2_kernel, n, nc, bc),
        grid=(nr, nc),
        in_specs=[
            pl.BlockSpec((br, bc), lambda r, c: (r, c)),
            pl.BlockSpec((br, 1), lambda r, c: (r, 0)),
            pl.BlockSpec((1, bc), lambda r, c: (0, c)),
            pl.BlockSpec((1, 1), lambda r, c: (0, 0)),
            pl.BlockSpec((1, 1), lambda r, c: (0, 0)),
            pl.BlockSpec((bc, h2f + 1), lambda r, c: (c, 0)),
            pl.BlockSpec((1, h2f), lambda r, c: (0, 0)),
            pl.BlockSpec((h2f, din), lambda r, c: (0, 0)),
            pl.BlockSpec((1, din), lambda r, c: (0, 0)),
        ],
        out_specs=pl.BlockSpec((br, din), lambda r, c: (r, 0)),
        out_shape=jax.ShapeDtypeStruct((n, din), jnp.float32),
        scratch_shapes=[
            pltpu.VMEM((br, h2f + 1), jnp.float32),
        ],
    )(adjb, s2, wj2.reshape(1, npad), ms2, mt2, whs2, sumwh2, w2_w,
      w2_b.reshape(1, din))

    return z
